# baseline (speedup 1.0000x reference)
import jax
import jax.numpy as jnp
from jax.experimental import pallas as pl
from jax.experimental.pallas import tpu as pltpu

_B, _N, _D, _H, _K = 16, 576, 384, 384, 5


def _dge_kernel(x_ref, diag_ref, w1_ref, b1_ref, w2_ref, b2_ref, o_ref):
    x = x_ref[0]
    norm = jnp.sqrt(jnp.sum(x * x, axis=1, keepdims=True))
    xn = x / (norm + 1e-8)
    s = jax.lax.dot_general(
        xn, xn, (((1,), (1,)), ((), ())), preferred_element_type=jnp.float32
    )
    s = s + diag_ref[...]
    neg_inf = jnp.float32(-jnp.inf)

    v1 = jnp.max(s, axis=1, keepdims=True)
    m = v1
    denom = jnp.ones_like(v1)
    for _ in range(_K - 1):
        m = jnp.max(jnp.where(s < m, s, neg_inf), axis=1, keepdims=True)
        denom = denom + jnp.exp(m - v1)

    p = jnp.where(s >= m, jnp.exp(s - v1), 0.0)
    agg = jnp.dot(p, x, preferred_element_type=jnp.float32) / denom
    h = x + agg
    h = jax.lax.dot_general(
        h, w1_ref[...], (((1,), (1,)), ((), ())),
        preferred_element_type=jnp.float32,
    )
    h = jnp.maximum(h + b1_ref[...], 0.0)
    h = jax.lax.dot_general(
        h, w2_ref[...], (((1,), (1,)), ((), ())),
        preferred_element_type=jnp.float32,
    )
    o_ref[0] = jnp.maximum(h + b2_ref[...], 0.0)


def kernel(x, W1, b1, W2, b2):
    b1r = b1.reshape(1, _H)
    b2r = b2.reshape(1, _H)
    diag = jnp.where(
        jax.lax.broadcasted_iota(jnp.int32, (_N, _N), 0)
        == jax.lax.broadcasted_iota(jnp.int32, (_N, _N), 1),
        jnp.float32(-1e30),
        jnp.float32(0.0),
    )
    out = pl.pallas_call(
        _dge_kernel,
        grid=(_B,),
        in_specs=[
            pl.BlockSpec((1, _N, _D), lambda b: (b, 0, 0)),
            pl.BlockSpec((_N, _N), lambda b: (0, 0)),
            pl.BlockSpec((_H, _D), lambda b: (0, 0)),
            pl.BlockSpec((1, _H), lambda b: (0, 0)),
            pl.BlockSpec((_H, _H), lambda b: (0, 0)),
            pl.BlockSpec((1, _H), lambda b: (0, 0)),
        ],
        out_specs=pl.BlockSpec((1, _N, _H), lambda b: (b, 0, 0)),
        out_shape=jax.ShapeDtypeStruct((_B, _N, _H), jnp.float32),
        compiler_params=pltpu.CompilerParams(
            dimension_semantics=("parallel",),
        ),
    )(x, diag, W1, b1r, W2, b2r)
    return out


# speedup vs baseline: 1.0548x; 1.0548x over previous
import jax
import jax.numpy as jnp
from jax.experimental import pallas as pl
from jax.experimental.pallas import tpu as pltpu

_B, _N, _D, _H, _K = 16, 576, 384, 384, 5


def _dge_kernel(x_ref, w1_ref, b1_ref, w2_ref, b2_ref, o_ref):
    x = x_ref[0]
    norm = jnp.sqrt(jnp.sum(x * x, axis=1, keepdims=True))
    xn = x / (norm + 1e-8)
    s = jax.lax.dot_general(
        xn, xn, (((1,), (1,)), ((), ())), preferred_element_type=jnp.float32
    )
    row = jax.lax.broadcasted_iota(jnp.int32, (_N, _N), 0)
    col = jax.lax.broadcasted_iota(jnp.int32, (_N, _N), 1)
    neg_inf = jnp.float32(-jnp.inf)
    s = jnp.where(row == col, neg_inf, s)

    v1 = jnp.max(s, axis=1, keepdims=True)
    m = v1
    denom = jnp.ones_like(v1)
    for _ in range(_K - 1):
        m = jnp.max(jnp.where(s < m, s, neg_inf), axis=1, keepdims=True)
        denom = denom + jnp.exp(m - v1)

    p = jnp.where(s >= m, jnp.exp(s - v1), 0.0)
    agg = jnp.dot(p, x, preferred_element_type=jnp.float32) / denom
    h = x + agg
    h = jax.lax.dot_general(
        h, w1_ref[...], (((1,), (1,)), ((), ())),
        preferred_element_type=jnp.float32,
    )
    h = jnp.maximum(h + b1_ref[...], 0.0)
    h = jax.lax.dot_general(
        h, w2_ref[...], (((1,), (1,)), ((), ())),
        preferred_element_type=jnp.float32,
    )
    o_ref[0] = jnp.maximum(h + b2_ref[...], 0.0)


def kernel(x, W1, b1, W2, b2):
    b1r = b1.reshape(1, _H)
    b2r = b2.reshape(1, _H)
    out = pl.pallas_call(
        _dge_kernel,
        grid=(_B,),
        in_specs=[
            pl.BlockSpec((1, _N, _D), lambda b: (b, 0, 0)),
            pl.BlockSpec((_H, _D), lambda b: (0, 0)),
            pl.BlockSpec((1, _H), lambda b: (0, 0)),
            pl.BlockSpec((_H, _H), lambda b: (0, 0)),
            pl.BlockSpec((1, _H), lambda b: (0, 0)),
        ],
        out_specs=pl.BlockSpec((1, _N, _H), lambda b: (b, 0, 0)),
        out_shape=jax.ShapeDtypeStruct((_B, _N, _H), jnp.float32),
        compiler_params=pltpu.CompilerParams(
            dimension_semantics=("parallel",),
        ),
    )(x, W1, b1r, W2, b2r)
    return out


# speedup vs baseline: 1.2371x; 1.1728x over previous
import jax
import jax.numpy as jnp
from jax.experimental import pallas as pl
from jax.experimental.pallas import tpu as pltpu

_B, _N, _D, _H, _K = 16, 576, 384, 384, 5


def _dge_kernel(x_ref, w1_ref, b1_ref, w2_ref, b2_ref, o_ref):
    x0 = x_ref[0]
    x1 = x_ref[1]
    norm0 = jnp.sqrt(jnp.sum(x0 * x0, axis=1, keepdims=True))
    norm1 = jnp.sqrt(jnp.sum(x1 * x1, axis=1, keepdims=True))
    xn0 = x0 / (norm0 + 1e-8)
    xn1 = x1 / (norm1 + 1e-8)
    s0 = jax.lax.dot_general(
        xn0, xn0, (((1,), (1,)), ((), ())), preferred_element_type=jnp.float32
    )
    s1 = jax.lax.dot_general(
        xn1, xn1, (((1,), (1,)), ((), ())), preferred_element_type=jnp.float32
    )
    row = jax.lax.broadcasted_iota(jnp.int32, (_N, _N), 0)
    col = jax.lax.broadcasted_iota(jnp.int32, (_N, _N), 1)
    diag = row == col
    neg_inf = jnp.float32(-jnp.inf)
    s0 = jnp.where(diag, neg_inf, s0)
    s1 = jnp.where(diag, neg_inf, s1)

    v10 = jnp.max(s0, axis=1, keepdims=True)
    v11 = jnp.max(s1, axis=1, keepdims=True)
    m0 = v10
    m1 = v11
    den0 = jnp.ones_like(v10)
    den1 = jnp.ones_like(v11)
    for _ in range(_K - 1):
        m0 = jnp.max(jnp.where(s0 < m0, s0, neg_inf), axis=1, keepdims=True)
        m1 = jnp.max(jnp.where(s1 < m1, s1, neg_inf), axis=1, keepdims=True)
        den0 = den0 + jnp.exp(m0 - v10)
        den1 = den1 + jnp.exp(m1 - v11)

    p0 = jnp.where(s0 >= m0, jnp.exp(s0 - v10), 0.0)
    p1 = jnp.where(s1 >= m1, jnp.exp(s1 - v11), 0.0)
    agg0 = jnp.dot(p0, x0, preferred_element_type=jnp.float32) / den0
    agg1 = jnp.dot(p1, x1, preferred_element_type=jnp.float32) / den1
    h0 = x0 + agg0
    h1 = x1 + agg1
    cdims = (((1,), (1,)), ((), ()))
    h0 = jax.lax.dot_general(
        h0, w1_ref[...], cdims, preferred_element_type=jnp.float32)
    h1 = jax.lax.dot_general(
        h1, w1_ref[...], cdims, preferred_element_type=jnp.float32)
    h0 = jnp.maximum(h0 + b1_ref[...], 0.0)
    h1 = jnp.maximum(h1 + b1_ref[...], 0.0)
    h0 = jax.lax.dot_general(
        h0, w2_ref[...], cdims, preferred_element_type=jnp.float32)
    h1 = jax.lax.dot_general(
        h1, w2_ref[...], cdims, preferred_element_type=jnp.float32)
    o_ref[0] = jnp.maximum(h0 + b2_ref[...], 0.0)
    o_ref[1] = jnp.maximum(h1 + b2_ref[...], 0.0)


def kernel(x, W1, b1, W2, b2):
    b1r = b1.reshape(1, _H)
    b2r = b2.reshape(1, _H)
    out = pl.pallas_call(
        _dge_kernel,
        grid=(_B // 2,),
        in_specs=[
            pl.BlockSpec((2, _N, _D), lambda b: (b, 0, 0)),
            pl.BlockSpec((_H, _D), lambda b: (0, 0)),
            pl.BlockSpec((1, _H), lambda b: (0, 0)),
            pl.BlockSpec((_H, _H), lambda b: (0, 0)),
            pl.BlockSpec((1, _H), lambda b: (0, 0)),
        ],
        out_specs=pl.BlockSpec((2, _N, _H), lambda b: (b, 0, 0)),
        out_shape=jax.ShapeDtypeStruct((_B, _N, _H), jnp.float32),
        compiler_params=pltpu.CompilerParams(
            dimension_semantics=("parallel",),
        ),
    )(x, W1, b1r, W2, b2r)
    return out
